# baseline (speedup 1.0000x reference)
import functools

import jax
import jax.numpy as jnp
from jax import lax
from jax.experimental import pallas as pl
from jax.experimental.pallas import tpu as pltpu
from jax.experimental.pallas import tpu_sc as plsc

NUM_CLASSES = 19
FULL_ROW, FULL_COL = 256, 512
NPIX = FULL_ROW * FULL_COL
C_FEAT = 512
NSAMP = 2048

BLK = 2048
NBLK = NPIX // BLK

NW = 32
SEG = NPIX // NW


def _dense_body(p_ref, f_ref, o_ref, out_ref):
    F = f_ref[...]
    O = o_ref[...]
    P = p_ref[...]

    G = jax.lax.dot_general(
        P.astype(jnp.bfloat16), F.astype(jnp.bfloat16),
        (((1,), (0,)), ((), ())),
        preferred_element_type=jnp.float32,
    )
    Fsq = F * F
    S = jax.lax.dot_general(
        jnp.ones((1, C_FEAT), jnp.float32), Fsq, (((1,), (0,)), ((), ())),
        preferred_element_type=jnp.float32,
        precision=jax.lax.Precision.HIGHEST,
    )
    pn = jnp.sum(P * P, axis=1, keepdims=True)

    d = S - 2.0 * G + pn
    d = jnp.maximum(d, 1e-12)

    m = jnp.max(O, axis=0, keepdims=True)
    K = jax.lax.broadcasted_iota(jnp.int32, (NUM_CLASSES, BLK), 0)
    cid = jnp.min(jnp.where(O == m, K, NUM_CLASSES), axis=0, keepdims=True)

    d_c = jnp.sum(jnp.where(K == cid, d, 0.0), axis=0, keepdims=True)
    before = (d < d_c) | ((d == d_c) & (K < cid))
    rank = jnp.sum(before.astype(jnp.float32), axis=0, keepdims=True)
    out_ref[0] = rank * (1.0 / (NUM_CLASSES - 1))


_dense_call = pl.pallas_call(
    _dense_body,
    grid=(NBLK,),
    in_specs=[
        pl.BlockSpec((NUM_CLASSES, C_FEAT), lambda i: (0, 0)),
        pl.BlockSpec((C_FEAT, BLK), lambda i: (0, i)),
        pl.BlockSpec((NUM_CLASSES, BLK), lambda i: (0, i)),
    ],
    out_specs=pl.BlockSpec((1, 1, BLK), lambda i: (i, 0, 0)),
    out_shape=jax.ShapeDtypeStruct((NBLK, 1, BLK), jnp.float32),
    compiler_params=pltpu.CompilerParams(
        dimension_semantics=("arbitrary",),
    ),
)


NTILE = 16
NPER = NSAMP // NTILE
HALF = NPIX // 2
ZSEG = HALF // NTILE
NDUMP = 16


@functools.lru_cache(maxsize=1)
def _make_scatter_call():
    @functools.partial(
        pl.kernel,
        mesh=plsc.VectorSubcoreMesh(core_axis_name="c", subcore_axis_name="s"),
        out_type=jax.ShapeDtypeStruct((NPIX + NDUMP,), jnp.float32),
        scratch_types=[
            pltpu.VMEM((1, NPER), jnp.int32),
            pltpu.VMEM((1, NPER), jnp.int32),
            pltpu.VMEM((NPER,), jnp.float32),
            pltpu.VMEM((ZSEG,), jnp.float32),
            pltpu.SemaphoreType.DMA,
        ],
    )
    def _scatter_call(idx_hbm, scores_hbm, out_hbm,
                      idx_v, loc_v, vals_v, zero_v, sem):
        c = lax.axis_index("c")
        s = lax.axis_index("s")

        def zero_body(i, carry):
            zero_v[pl.ds(i * 16, 16)] = jnp.zeros((16,), jnp.float32)
            return carry

        lax.fori_loop(0, ZSEG // 16, zero_body, 0)
        zbase = c * HALF + s * ZSEG
        pltpu.sync_copy(zero_v, out_hbm.at[pl.ds(zbase, ZSEG)])
        plsc.subcore_barrier()

        pltpu.sync_copy(idx_hbm.at[s], idx_v.at[0])
        pltpu.async_copy(scores_hbm.at[idx_v.at[0]], vals_v, sem).wait()
        half_lo = c * HALF
        lanes = lax.iota(jnp.int32, 16)
        for j in range(NPER // 16):
            iv = idx_v[0, pl.ds(j * 16, 16)]
            mine = (iv >= half_lo) & (iv < half_lo + HALF)
            loc_v[0, pl.ds(j * 16, 16)] = jnp.where(mine, iv, NPIX + lanes)
        pltpu.async_copy(vals_v, out_hbm.at[loc_v.at[0]], sem).wait()

    return _scatter_call


def kernel(features_tensor, outputs, classes_prototypes, sample_index):
    F = features_tensor.reshape(C_FEAT, NPIX)
    O = outputs.reshape(NUM_CLASSES, NPIX)
    P = classes_prototypes.reshape(NUM_CLASSES, C_FEAT)
    idx = sample_index.astype(jnp.int32)

    scores = _dense_call(P, F, O).reshape(NPIX)
    full = _make_scatter_call()(idx.reshape(NTILE, NPER), scores)
    return full[:NPIX].reshape(FULL_ROW, FULL_COL)


# speedup vs baseline: 6.1609x; 6.1609x over previous
import functools

import jax
import jax.numpy as jnp
from jax import lax
from jax.experimental import pallas as pl
from jax.experimental.pallas import tpu as pltpu
from jax.experimental.pallas import tpu_sc as plsc

NUM_CLASSES = 19
FULL_ROW, FULL_COL = 256, 512
NPIX = FULL_ROW * FULL_COL
C_FEAT = 512
NSAMP = 2048

BR = 16
NBLK = FULL_ROW // BR
NW = 32
SPW = NSAMP // NW


def _dense_body(p_ref, f_ref, o_ref, out_ref):
    P = p_ref[...]
    Pb = P.astype(jnp.bfloat16)
    pn = jnp.sum(P * P, axis=1, keepdims=True)
    K = lax.broadcasted_iota(jnp.int32, (NUM_CLASSES, FULL_COL), 0)
    Ft = jnp.transpose(f_ref[...].astype(jnp.bfloat16), (1, 0, 2))
    for r in range(BR):
        Or = o_ref[:, r, :]
        G = lax.dot_general(
            Pb, Ft[r], (((1,), (0,)), ((), ())),
            preferred_element_type=jnp.float32,
        )
        d = pn - 2.0 * G
        m = jnp.max(Or, axis=0, keepdims=True)
        cid = jnp.min(jnp.where(Or == m, K, NUM_CLASSES), axis=0,
                      keepdims=True)
        d_c = jnp.sum(jnp.where(K == cid, d, 0.0), axis=0, keepdims=True)
        before = (d < d_c) | ((d == d_c) & (K < cid))
        rank = jnp.sum(before.astype(jnp.float32), axis=0, keepdims=True)
        out_ref[0, :, pl.ds(r * FULL_COL, FULL_COL)] = (
            rank * (1.0 / (NUM_CLASSES - 1)))


_dense_call = pl.pallas_call(
    _dense_body,
    grid=(NBLK,),
    in_specs=[
        pl.BlockSpec((NUM_CLASSES, C_FEAT), lambda i: (0, 0)),
        pl.BlockSpec((C_FEAT, BR, FULL_COL), lambda i: (0, i, 0)),
        pl.BlockSpec((NUM_CLASSES, BR, FULL_COL), lambda i: (0, i, 0)),
    ],
    out_specs=pl.BlockSpec((1, 1, BR * FULL_COL), lambda i: (i, 0, 0)),
    out_shape=jax.ShapeDtypeStruct((NBLK, 1, BR * FULL_COL), jnp.float32),
    compiler_params=pltpu.CompilerParams(
        dimension_semantics=("parallel",),
    ),
)


@functools.lru_cache(maxsize=1)
def _make_scatter_call():
    @functools.partial(
        pl.kernel,
        mesh=plsc.VectorSubcoreMesh(core_axis_name="c", subcore_axis_name="s"),
        out_type=(),
        scratch_types=[
            pltpu.VMEM((1, SPW), jnp.int32),
            pltpu.VMEM((SPW,), jnp.float32),
            pltpu.SemaphoreType.DMA,
        ],
    )
    def _scatter_call(idx_hbm, scores_hbm, out_ref, idx_v, vals_v, sem):
        wid = lax.axis_index("s") * 2 + lax.axis_index("c")
        pltpu.sync_copy(idx_hbm.at[wid], idx_v.at[0])
        pltpu.async_copy(scores_hbm.at[idx_v.at[0]], vals_v, sem).wait()
        pltpu.async_copy(vals_v, out_ref.at[idx_v.at[0]], sem).wait()

    return _scatter_call


def kernel(features_tensor, outputs, classes_prototypes, sample_index):
    F = features_tensor.reshape(C_FEAT, FULL_ROW, FULL_COL)
    O = outputs.reshape(NUM_CLASSES, FULL_ROW, FULL_COL)
    P = classes_prototypes.reshape(NUM_CLASSES, C_FEAT)
    idx = sample_index.astype(jnp.int32)

    scores = _dense_call(P, F, O).reshape(NPIX)
    full_ref = jax.new_ref(jnp.zeros((NPIX,), jnp.float32))
    _make_scatter_call()(idx.reshape(NW, SPW), scores, full_ref)
    return jax.freeze(full_ref).reshape(FULL_ROW, FULL_COL)
